# Initial kernel scaffold; baseline (speedup 1.0000x reference)
import jax
import jax.numpy as jnp
from jax.experimental import pallas as pl


def kernel(x, edge_index, W1, att_src1, att_dst1, b1, W2, att_src2, att_dst2, b2):
    raise NotImplementedError("write your pallas kernel here")


# speedup vs baseline: 26.2966x; 26.2966x over previous
import functools

import jax
import jax.numpy as jnp
from jax import lax
from jax.experimental import pallas as pl
from jax.experimental.pallas import tpu as pltpu
from jax.experimental.pallas import tpu_sc as plsc

F32 = jnp.float32
_PREC = lax.Precision.HIGHEST

_NC = 2
_NS = 16
_NW = _NC * _NS
_K = 80


def _splat(v, lane):
    idx = jnp.full((16, 1), lane, jnp.int32)
    dn = lax.GatherDimensionNumbers(
        offset_dims=(), collapsed_slice_dims=(0,), start_index_map=(0,))
    return lax.gather(v, idx, dn, (1,),
                      mode=lax.GatherScatterMode.PROMISE_IN_BOUNDS)


def _make_edge_kernel(n, npad, e, d, nheads):
    ept = e // _NW
    c_chunks = ept // _K
    nj = d // 16
    rpt = npad // _NS

    mesh = plsc.VectorSubcoreMesh(core_axis_name="c", subcore_axis_name="s",
                                  num_cores=_NC, num_subcores=_NS)

    def body(src_h, dst_h, tab_h, as_h, ad_h, zd_h, z16_h, outp, denp,
             idxs, idxd, ar, br, hr, wb, msg, acc, dacc, sem):
        cid = lax.axis_index("c")
        sid = lax.axis_index("s")
        wid = sid * _NC + cid

        zsl = pl.ds(sid * rpt, rpt)
        pltpu.sync_copy(zd_h, acc.at[zsl])
        pltpu.sync_copy(z16_h, dacc.at[zsl])
        plsc.subcore_barrier()

        def chunk(ci, _):
            base = wid * ept + ci * _K
            pltpu.sync_copy(src_h.at[pl.ds(base, _K)], idxs)
            pltpu.sync_copy(dst_h.at[pl.ds(base, _K)], idxd)
            pltpu.async_copy(tab_h.at[idxs], hr, sem).wait()
            pltpu.async_copy(as_h.at[idxs], ar, sem).wait()
            pltpu.async_copy(ad_h.at[idxd], br, sem).wait()

            def edge(ei, _):
                s = ar[ei, :] + br[ei, :]
                w = jnp.exp(jnp.maximum(s, 0.2 * s))
                wb[ei, :] = w
                for j in range(nj):
                    sp = _splat(w, j if nheads > 1 else 0)
                    fsl = pl.ds(16 * j, 16)
                    msg[ei, fsl] = hr[ei, fsl] * sp
                return 0

            lax.fori_loop(0, _K, edge, 0)
            pltpu.sync_copy(wb, dacc.at[idxd], add=True)
            pltpu.sync_copy(msg, acc.at[idxd], add=True)
            return 0

        lax.fori_loop(0, c_chunks, chunk, 0)
        plsc.subcore_barrier()

        sl = pl.ds(sid * rpt, rpt)
        pltpu.sync_copy(acc.at[sl], outp.at[cid, sl])
        pltpu.sync_copy(dacc.at[sl], denp.at[cid, sl])

    return pl.kernel(
        body,
        out_type=[jax.ShapeDtypeStruct((_NC, npad, d), F32),
                  jax.ShapeDtypeStruct((_NC, npad, 16), F32)],
        mesh=mesh,
        compiler_params=pltpu.CompilerParams(use_tc_tiling_on_sc=False),
        scratch_types=[
            pltpu.VMEM((_K,), jnp.int32),
            pltpu.VMEM((_K,), jnp.int32),
            pltpu.VMEM((_K, 16), F32),
            pltpu.VMEM((_K, 16), F32),
            pltpu.VMEM((_K, d), F32),
            pltpu.VMEM((_K, 16), F32),
            pltpu.VMEM((_K, d), F32),
            pltpu.VMEM_SHARED((npad, d), F32),
            pltpu.VMEM_SHARED((npad, 16), F32),
            pltpu.SemaphoreType.DMA,
        ],
    )


def _tc_a(x_ref, w1_ref, as_ref, ad_ref, h_ref, a1_ref, a2_ref):
    h = jnp.dot(x_ref[...], w1_ref[...], precision=_PREC)
    h_ref[...] = h
    a1_ref[...] = jnp.dot(h, as_ref[...], precision=_PREC)
    a2_ref[...] = jnp.dot(h, ad_ref[...], precision=_PREC)


def _tc_b(o_ref, d_ref, r_ref, b1_ref, w2_ref, as_ref, ad_ref,
          h2_ref, a1_ref, a2_ref):
    o = o_ref[0] + o_ref[1]
    den = d_ref[0] + d_ref[1]
    denr = jnp.dot(den, r_ref[...], precision=_PREC) + 1e-16
    oo = o / denr + b1_ref[...]
    hf = jnp.where(oo > 0, oo, jnp.exp(oo) - 1.0)
    h2 = jnp.dot(hf, w2_ref[...], precision=_PREC)
    h2_ref[...] = h2
    a1_ref[...] = jnp.dot(h2, as_ref[...], precision=_PREC)
    a2_ref[...] = jnp.dot(h2, ad_ref[...], precision=_PREC)


def _tc_c(o_ref, d_ref, r_ref, b2_ref, out_ref):
    o = o_ref[0] + o_ref[1]
    den = d_ref[0] + d_ref[1]
    denr = jnp.dot(den, r_ref[...], precision=_PREC) + 1e-16
    out_ref[...] = o / denr + b2_ref[...]


def kernel(x, edge_index, W1, att_src1, att_dst1, b1, W2, att_src2,
           att_dst2, b2):
    n, d_in = x.shape
    e = edge_index.shape[1]
    d1 = W1.shape[1]
    heads = att_src1.shape[0]
    hid = att_src1.shape[1]
    d2 = W2.shape[1]

    src = edge_index[0]
    dst = edge_index[1]

    eye = jnp.eye(heads, dtype=F32)
    As1 = jnp.pad((att_src1[:, :, None] * eye[:, None, :]).reshape(d1, heads),
                  ((0, 0), (0, 16 - heads)))
    Ad1 = jnp.pad((att_dst1[:, :, None] * eye[:, None, :]).reshape(d1, heads),
                  ((0, 0), (0, 16 - heads)))
    As2 = jnp.pad(att_src2.T, ((0, 0), (0, 15)))
    Ad2 = jnp.pad(att_dst2.T, ((0, 0), (0, 15)))
    R1 = jnp.pad(jnp.repeat(jnp.eye(heads, dtype=F32), hid, axis=1),
                 ((0, 16 - heads), (0, 0)))
    R2 = jnp.zeros((16, d2), F32).at[0].set(1.0)

    npad = ((n + 8 * _NS - 1) // (8 * _NS)) * (8 * _NS)
    rpt = npad // _NS
    zd1 = jnp.zeros((rpt, d1), F32)
    zd2 = jnp.zeros((rpt, d2), F32)
    z16 = jnp.zeros((rpt, 16), F32)

    bn = 1000
    grid = (n // bn,)

    h1, as1, ad1 = pl.pallas_call(
        _tc_a,
        grid=grid,
        in_specs=[pl.BlockSpec((bn, d_in), lambda i: (i, 0)),
                  pl.BlockSpec((d_in, d1), lambda i: (0, 0)),
                  pl.BlockSpec((d1, 16), lambda i: (0, 0)),
                  pl.BlockSpec((d1, 16), lambda i: (0, 0))],
        out_specs=[pl.BlockSpec((bn, d1), lambda i: (i, 0)),
                   pl.BlockSpec((bn, 16), lambda i: (i, 0)),
                   pl.BlockSpec((bn, 16), lambda i: (i, 0))],
        out_shape=[jax.ShapeDtypeStruct((n, d1), F32),
                   jax.ShapeDtypeStruct((n, 16), F32),
                   jax.ShapeDtypeStruct((n, 16), F32)],
    )(x, W1, As1, Ad1)

    o1p, d1p = _make_edge_kernel(n, npad, e, d1, heads)(
        src, dst, h1, as1, ad1, zd1, z16)

    h2, as2, ad2 = pl.pallas_call(
        _tc_b,
        grid=grid,
        in_specs=[pl.BlockSpec((_NC, bn, d1), lambda i: (0, i, 0)),
                  pl.BlockSpec((_NC, bn, 16), lambda i: (0, i, 0)),
                  pl.BlockSpec((16, d1), lambda i: (0, 0)),
                  pl.BlockSpec((1, d1), lambda i: (0, 0)),
                  pl.BlockSpec((d1, d2), lambda i: (0, 0)),
                  pl.BlockSpec((d2, 16), lambda i: (0, 0)),
                  pl.BlockSpec((d2, 16), lambda i: (0, 0))],
        out_specs=[pl.BlockSpec((bn, d2), lambda i: (i, 0)),
                   pl.BlockSpec((bn, 16), lambda i: (i, 0)),
                   pl.BlockSpec((bn, 16), lambda i: (i, 0))],
        out_shape=[jax.ShapeDtypeStruct((n, d2), F32),
                   jax.ShapeDtypeStruct((n, 16), F32),
                   jax.ShapeDtypeStruct((n, 16), F32)],
    )(o1p, d1p, R1, b1.reshape(1, d1), W2, As2, Ad2)

    o2p, d2p = _make_edge_kernel(n, npad, e, d2, 1)(
        src, dst, h2, as2, ad2, zd2, z16)

    out = pl.pallas_call(
        _tc_c,
        grid=grid,
        in_specs=[pl.BlockSpec((_NC, bn, d2), lambda i: (0, i, 0)),
                  pl.BlockSpec((_NC, bn, 16), lambda i: (0, i, 0)),
                  pl.BlockSpec((16, d2), lambda i: (0, 0)),
                  pl.BlockSpec((1, d2), lambda i: (0, 0))],
        out_specs=pl.BlockSpec((bn, d2), lambda i: (i, 0)),
        out_shape=jax.ShapeDtypeStruct((n, d2), F32),
    )(o2p, d2p, R2, b2.reshape(1, d2))

    return out


# speedup vs baseline: 94.7580x; 3.6034x over previous
import functools

import jax
import jax.numpy as jnp
from jax import lax
from jax.experimental import pallas as pl
from jax.experimental.pallas import tpu as pltpu
from jax.experimental.pallas import tpu_sc as plsc

F32 = jnp.float32
_PREC = lax.Precision.HIGHEST

_NC = 2
_NS = 16
_NW = _NC * _NS
_K = 80


def _splat(v, lane):
    idx = jnp.full((16, 1), lane, jnp.int32)
    dn = lax.GatherDimensionNumbers(
        offset_dims=(), collapsed_slice_dims=(0,), start_index_map=(0,))
    return lax.gather(v, idx, dn, (1,),
                      mode=lax.GatherScatterMode.PROMISE_IN_BOUNDS)


def _make_edge_kernel(n, npad, e, d, nheads):
    ept = e // _NW
    c_chunks = ept // _K
    nj = d // 16
    rpt = npad // _NS

    mesh = plsc.VectorSubcoreMesh(core_axis_name="c", subcore_axis_name="s",
                                  num_cores=_NC, num_subcores=_NS)

    def body(src_h, dst_h, tab_h, as_h, ad_h, zd_h, z16_h, outp, denp,
             idxs0, idxd0, ar0, br0, hr0,
             idxs1, idxd1, ar1, br1, hr1, wb, msg,
             acc, dacc, semg0, semg1, semi0, semi1):
        cid = lax.axis_index("c")
        sid = lax.axis_index("s")
        wid = sid * _NC + cid

        bufs = ((idxs0, idxd0, ar0, br0, hr0, wb, msg, semg0, semi0),
                (idxs1, idxd1, ar1, br1, hr1, wb, msg, semg1, semi1))

        zsl = pl.ds(sid * rpt, rpt)
        pltpu.sync_copy(zd_h, acc.at[zsl])
        pltpu.sync_copy(z16_h, dacc.at[zsl])
        plsc.subcore_barrier()

        def issue_idx(ci, b, sync=False):
            idxs, idxd = b[0], b[1]
            base = wid * ept + ci * _K
            if sync:
                pltpu.sync_copy(src_h.at[pl.ds(base, _K)], idxs)
                pltpu.sync_copy(dst_h.at[pl.ds(base, _K)], idxd)
            else:
                pltpu.async_copy(src_h.at[pl.ds(base, _K)], idxs, b[8])
                pltpu.async_copy(dst_h.at[pl.ds(base, _K)], idxd, b[8])

        def wait_idx(b):
            pltpu.make_async_copy(src_h.at[b[0]], b[0], b[8]).wait()
            pltpu.make_async_copy(dst_h.at[b[1]], b[1], b[8]).wait()

        def issue_gathers(b):
            idxs, idxd, ar, br, hr = b[0], b[1], b[2], b[3], b[4]
            pltpu.async_copy(tab_h.at[idxs], hr, b[7])
            pltpu.async_copy(as_h.at[idxs], ar, b[7])
            pltpu.async_copy(ad_h.at[idxd], br, b[7])

        def wait_gathers(b):
            pltpu.make_async_copy(tab_h.at[b[0]], b[4], b[7]).wait()
            pltpu.make_async_copy(as_h.at[b[0]], b[2], b[7]).wait()
            pltpu.make_async_copy(ad_h.at[b[1]], b[3], b[7]).wait()

        def step(ci, b, bn):
            idxd, ar, br, hr, wb, msg = b[1], b[2], b[3], b[4], b[5], b[6]
            wait_gathers(b)

            @pl.when(ci + 1 < c_chunks)
            def _():
                wait_idx(bn)
                issue_gathers(bn)

            @plsc.parallel_loop(0, _K, 1, unroll=8)
            def _(ei):
                s = ar[ei, :] + br[ei, :]
                w = jnp.exp(jnp.maximum(s, 0.2 * s))
                wb[ei, :] = w
                for j in range(nj):
                    sp = _splat(w, j if nheads > 1 else 0)
                    fsl = pl.ds(16 * j, 16)
                    msg[ei, fsl] = hr[ei, fsl] * sp

            pltpu.sync_copy(wb, dacc.at[idxd], add=True)
            pltpu.sync_copy(msg, acc.at[idxd], add=True)

            @pl.when(ci + 2 < c_chunks)
            def _():
                issue_idx(ci + 2, b)

        issue_idx(0, bufs[0], sync=True)
        issue_gathers(bufs[0])
        issue_idx(1, bufs[1])

        def pair(j, _):
            step(2 * j, bufs[0], bufs[1])
            step(2 * j + 1, bufs[1], bufs[0])
            return 0

        lax.fori_loop(0, c_chunks // 2, pair, 0)
        if c_chunks % 2:
            step(c_chunks - 1, bufs[0], bufs[1])

        plsc.subcore_barrier()

        sl = pl.ds(sid * rpt, rpt)
        pltpu.sync_copy(acc.at[sl], outp.at[cid, sl])
        pltpu.sync_copy(dacc.at[sl], denp.at[cid, sl])

    return pl.kernel(
        body,
        out_type=[jax.ShapeDtypeStruct((_NC, npad, d), F32),
                  jax.ShapeDtypeStruct((_NC, npad, 16), F32)],
        mesh=mesh,
        compiler_params=pltpu.CompilerParams(use_tc_tiling_on_sc=False),
        scratch_types=(
            [pltpu.VMEM((_K,), jnp.int32),
             pltpu.VMEM((_K,), jnp.int32),
             pltpu.VMEM((_K, 16), F32),
             pltpu.VMEM((_K, 16), F32),
             pltpu.VMEM((_K, d), F32)] * 2 +
            [pltpu.VMEM((_K, 16), F32),
             pltpu.VMEM((_K, d), F32),
             pltpu.VMEM_SHARED((npad, d), F32),
             pltpu.VMEM_SHARED((npad, 16), F32),
             pltpu.SemaphoreType.DMA,
             pltpu.SemaphoreType.DMA,
             pltpu.SemaphoreType.DMA,
             pltpu.SemaphoreType.DMA]
        ),
    )


def _tc_a(x_ref, w1_ref, as_ref, ad_ref, h_ref, a1_ref, a2_ref):
    h = jnp.dot(x_ref[...], w1_ref[...], precision=_PREC)
    h_ref[...] = h
    a1_ref[...] = jnp.dot(h, as_ref[...], precision=_PREC)
    a2_ref[...] = jnp.dot(h, ad_ref[...], precision=_PREC)


def _tc_b(o_ref, d_ref, r_ref, b1_ref, w2_ref, as_ref, ad_ref,
          h2_ref, a1_ref, a2_ref):
    o = o_ref[0] + o_ref[1]
    den = d_ref[0] + d_ref[1]
    denr = jnp.dot(den, r_ref[...], precision=_PREC) + 1e-16
    oo = o / denr + b1_ref[...]
    hf = jnp.where(oo > 0, oo, jnp.exp(oo) - 1.0)
    h2 = jnp.dot(hf, w2_ref[...], precision=_PREC)
    h2_ref[...] = h2
    a1_ref[...] = jnp.dot(h2, as_ref[...], precision=_PREC)
    a2_ref[...] = jnp.dot(h2, ad_ref[...], precision=_PREC)


def _tc_c(o_ref, d_ref, r_ref, b2_ref, out_ref):
    o = o_ref[0] + o_ref[1]
    den = d_ref[0] + d_ref[1]
    denr = jnp.dot(den, r_ref[...], precision=_PREC) + 1e-16
    out_ref[...] = o / denr + b2_ref[...]


def kernel(x, edge_index, W1, att_src1, att_dst1, b1, W2, att_src2,
           att_dst2, b2):
    n, d_in = x.shape
    e = edge_index.shape[1]
    d1 = W1.shape[1]
    heads = att_src1.shape[0]
    hid = att_src1.shape[1]
    d2 = W2.shape[1]

    src = edge_index[0]
    dst = edge_index[1]

    eye = jnp.eye(heads, dtype=F32)
    As1 = jnp.pad((att_src1[:, :, None] * eye[:, None, :]).reshape(d1, heads),
                  ((0, 0), (0, 16 - heads)))
    Ad1 = jnp.pad((att_dst1[:, :, None] * eye[:, None, :]).reshape(d1, heads),
                  ((0, 0), (0, 16 - heads)))
    As2 = jnp.pad(att_src2.T, ((0, 0), (0, 15)))
    Ad2 = jnp.pad(att_dst2.T, ((0, 0), (0, 15)))
    R1 = jnp.pad(jnp.repeat(jnp.eye(heads, dtype=F32), hid, axis=1),
                 ((0, 16 - heads), (0, 0)))
    R2 = jnp.zeros((16, d2), F32).at[0].set(1.0)

    npad = ((n + 8 * _NS - 1) // (8 * _NS)) * (8 * _NS)
    rpt = npad // _NS
    zd1 = jnp.zeros((rpt, d1), F32)
    zd2 = jnp.zeros((rpt, d2), F32)
    z16 = jnp.zeros((rpt, 16), F32)

    bn = 1000
    grid = (n // bn,)

    h1, as1, ad1 = pl.pallas_call(
        _tc_a,
        grid=grid,
        in_specs=[pl.BlockSpec((bn, d_in), lambda i: (i, 0)),
                  pl.BlockSpec((d_in, d1), lambda i: (0, 0)),
                  pl.BlockSpec((d1, 16), lambda i: (0, 0)),
                  pl.BlockSpec((d1, 16), lambda i: (0, 0))],
        out_specs=[pl.BlockSpec((bn, d1), lambda i: (i, 0)),
                   pl.BlockSpec((bn, 16), lambda i: (i, 0)),
                   pl.BlockSpec((bn, 16), lambda i: (i, 0))],
        out_shape=[jax.ShapeDtypeStruct((n, d1), F32),
                   jax.ShapeDtypeStruct((n, 16), F32),
                   jax.ShapeDtypeStruct((n, 16), F32)],
    )(x, W1, As1, Ad1)

    o1p, d1p = _make_edge_kernel(n, npad, e, d1, heads)(
        src, dst, h1, as1, ad1, zd1, z16)

    h2, as2, ad2 = pl.pallas_call(
        _tc_b,
        grid=grid,
        in_specs=[pl.BlockSpec((_NC, bn, d1), lambda i: (0, i, 0)),
                  pl.BlockSpec((_NC, bn, 16), lambda i: (0, i, 0)),
                  pl.BlockSpec((16, d1), lambda i: (0, 0)),
                  pl.BlockSpec((1, d1), lambda i: (0, 0)),
                  pl.BlockSpec((d1, d2), lambda i: (0, 0)),
                  pl.BlockSpec((d2, 16), lambda i: (0, 0)),
                  pl.BlockSpec((d2, 16), lambda i: (0, 0))],
        out_specs=[pl.BlockSpec((bn, d2), lambda i: (i, 0)),
                   pl.BlockSpec((bn, 16), lambda i: (i, 0)),
                   pl.BlockSpec((bn, 16), lambda i: (i, 0))],
        out_shape=[jax.ShapeDtypeStruct((n, d2), F32),
                   jax.ShapeDtypeStruct((n, 16), F32),
                   jax.ShapeDtypeStruct((n, 16), F32)],
    )(o1p, d1p, R1, b1.reshape(1, d1), W2, As2, Ad2)

    o2p, d2p = _make_edge_kernel(n, npad, e, d2, 1)(
        src, dst, h2, as2, ad2, zd2, z16)

    out = pl.pallas_call(
        _tc_c,
        grid=grid,
        in_specs=[pl.BlockSpec((_NC, bn, d2), lambda i: (0, i, 0)),
                  pl.BlockSpec((_NC, bn, 16), lambda i: (0, i, 0)),
                  pl.BlockSpec((16, d2), lambda i: (0, 0)),
                  pl.BlockSpec((1, d2), lambda i: (0, 0))],
        out_specs=pl.BlockSpec((bn, d2), lambda i: (i, 0)),
        out_shape=jax.ShapeDtypeStruct((n, d2), F32),
    )(o2p, d2p, R2, b2.reshape(1, d2))

    return out


# speedup vs baseline: 116.1892x; 1.2262x over previous
import functools

import jax
import jax.numpy as jnp
from jax import lax
from jax.experimental import pallas as pl
from jax.experimental.pallas import tpu as pltpu
from jax.experimental.pallas import tpu_sc as plsc

F32 = jnp.float32
_PREC = lax.Precision.DEFAULT

_NC = 2
_NS = 16
_NW = _NC * _NS
_K = 80


def _splat(v, lane):
    idx = jnp.full((16, 1), lane, jnp.int32)
    dn = lax.GatherDimensionNumbers(
        offset_dims=(), collapsed_slice_dims=(0,), start_index_map=(0,))
    return lax.gather(v, idx, dn, (1,),
                      mode=lax.GatherScatterMode.PROMISE_IN_BOUNDS)


def _make_edge_kernel(n, npad, e, d, nheads):
    ept = e // _NW
    c_chunks = ept // _K
    nj = d // 16
    rpt = npad // _NS

    mesh = plsc.VectorSubcoreMesh(core_axis_name="c", subcore_axis_name="s",
                                  num_cores=_NC, num_subcores=_NS)

    def body(ei_h, tab_h, as_h, ad_h, zd_h, z16_h, outp, denp,
             idxs0, idxd0, ar0, br0, hr0,
             idxs1, idxd1, ar1, br1, hr1, wb, msg,
             acc, dacc, semg0, semg1, semi0, semi1, sems):
        cid = lax.axis_index("c")
        sid = lax.axis_index("s")
        wid = sid * _NC + cid

        bufs = ((idxs0, idxd0, ar0, br0, hr0, wb, msg, semg0, semi0),
                (idxs1, idxd1, ar1, br1, hr1, wb, msg, semg1, semi1))

        zsl = pl.ds(sid * rpt, rpt)
        pltpu.sync_copy(zd_h, acc.at[zsl])
        pltpu.sync_copy(z16_h, dacc.at[zsl])
        plsc.subcore_barrier()

        def issue_idx(ci, b, sync=False):
            idxs, idxd = b[0], b[1]
            base = wid * ept + ci * _K
            if sync:
                pltpu.sync_copy(ei_h.at[0, pl.ds(base, _K)], idxs)
                pltpu.sync_copy(ei_h.at[1, pl.ds(base, _K)], idxd)
            else:
                pltpu.async_copy(ei_h.at[0, pl.ds(base, _K)], idxs, b[8])
                pltpu.async_copy(ei_h.at[1, pl.ds(base, _K)], idxd, b[8])

        def wait_idx(b):
            pltpu.make_async_copy(ei_h.at[0, pl.ds(0, _K)], b[0], b[8]).wait()
            pltpu.make_async_copy(ei_h.at[1, pl.ds(0, _K)], b[1], b[8]).wait()

        def issue_gathers(b):
            idxs, idxd, ar, br, hr = b[0], b[1], b[2], b[3], b[4]
            pltpu.async_copy(tab_h.at[idxs], hr, b[7])
            pltpu.async_copy(as_h.at[idxs], ar, b[7])
            pltpu.async_copy(ad_h.at[idxd], br, b[7])

        def wait_gathers(b):
            pltpu.make_async_copy(tab_h.at[b[0]], b[4], b[7]).wait()
            pltpu.make_async_copy(as_h.at[b[0]], b[2], b[7]).wait()
            pltpu.make_async_copy(ad_h.at[b[1]], b[3], b[7]).wait()

        def wait_scatter(b):
            pltpu.make_async_copy(b[5], dacc.at[b[1]], sems).wait()
            pltpu.make_async_copy(b[6], acc.at[b[1]], sems).wait()

        def step(ci, b, bn):
            idxd, ar, br, hr, wb, msg = b[1], b[2], b[3], b[4], b[5], b[6]
            wait_gathers(b)

            @pl.when(ci + 1 < c_chunks)
            def _():
                wait_idx(bn)
                issue_gathers(bn)

            @pl.when(ci > 0)
            def _():
                wait_scatter(b)

            @plsc.parallel_loop(0, _K, 1, unroll=8)
            def _(ei):
                s = ar[ei, :] + br[ei, :]
                w = jnp.exp(jnp.maximum(s, 0.2 * s))
                wb[ei, :] = w
                for j in range(nj):
                    sp = _splat(w, j if nheads > 1 else 0)
                    fsl = pl.ds(16 * j, 16)
                    msg[ei, fsl] = hr[ei, fsl] * sp

            pltpu.async_copy(wb, dacc.at[idxd], sems, add=True)
            pltpu.async_copy(msg, acc.at[idxd], sems, add=True)

            @pl.when(ci + 2 < c_chunks)
            def _():
                issue_idx(ci + 2, b)

        issue_idx(0, bufs[0], sync=True)
        issue_gathers(bufs[0])
        issue_idx(1, bufs[1])

        def pair(j, _):
            step(2 * j, bufs[0], bufs[1])
            step(2 * j + 1, bufs[1], bufs[0])
            return 0

        lax.fori_loop(0, c_chunks // 2, pair, 0)
        if c_chunks % 2:
            step(c_chunks - 1, bufs[0], bufs[1])

        wait_scatter(bufs[0] if c_chunks % 2 else bufs[1])
        plsc.subcore_barrier()

        sl = pl.ds(sid * rpt, rpt)
        pltpu.sync_copy(acc.at[sl], outp.at[cid, sl])
        pltpu.sync_copy(dacc.at[sl], denp.at[cid, sl])

    return pl.kernel(
        body,
        out_type=[jax.ShapeDtypeStruct((_NC, npad, d), F32),
                  jax.ShapeDtypeStruct((_NC, npad, 16), F32)],
        mesh=mesh,
        compiler_params=pltpu.CompilerParams(use_tc_tiling_on_sc=False),
        scratch_types=(
            [pltpu.VMEM((_K,), jnp.int32),
             pltpu.VMEM((_K,), jnp.int32),
             pltpu.VMEM((_K, 16), F32),
             pltpu.VMEM((_K, 16), F32),
             pltpu.VMEM((_K, d), F32)] * 2 +
            [pltpu.VMEM((_K, 16), F32),
             pltpu.VMEM((_K, d), F32),
             pltpu.VMEM_SHARED((npad, d), F32),
             pltpu.VMEM_SHARED((npad, 16), F32),
             pltpu.SemaphoreType.DMA,
             pltpu.SemaphoreType.DMA,
             pltpu.SemaphoreType.DMA,
             pltpu.SemaphoreType.DMA,
             pltpu.SemaphoreType.DMA]
        ),
    )


def _tc_a(x_ref, w1_ref, as_ref, ad_ref, h_ref, a1_ref, a2_ref):
    h = jnp.dot(x_ref[...], w1_ref[...], precision=_PREC)
    h_ref[...] = h
    a1_ref[...] = jnp.dot(h, as_ref[...], precision=_PREC)
    a2_ref[...] = jnp.dot(h, ad_ref[...], precision=_PREC)


def _tc_b(o_ref, d_ref, r_ref, b1_ref, w2_ref, as_ref, ad_ref,
          h2_ref, a1_ref, a2_ref):
    o = o_ref[0] + o_ref[1]
    den = d_ref[0] + d_ref[1]
    denr = jnp.dot(den, r_ref[...], precision=_PREC) + 1e-16
    oo = o / denr + b1_ref[...]
    hf = jnp.where(oo > 0, oo, jnp.exp(oo) - 1.0)
    h2 = jnp.dot(hf, w2_ref[...], precision=_PREC)
    h2_ref[...] = h2
    a1_ref[...] = jnp.dot(h2, as_ref[...], precision=_PREC)
    a2_ref[...] = jnp.dot(h2, ad_ref[...], precision=_PREC)


def _tc_c(o_ref, d_ref, r_ref, b2_ref, out_ref):
    o = o_ref[0] + o_ref[1]
    den = d_ref[0] + d_ref[1]
    denr = jnp.dot(den, r_ref[...], precision=_PREC) + 1e-16
    out_ref[...] = o / denr + b2_ref[...]


def kernel(x, edge_index, W1, att_src1, att_dst1, b1, W2, att_src2,
           att_dst2, b2):
    n, d_in = x.shape
    e = edge_index.shape[1]
    d1 = W1.shape[1]
    heads = att_src1.shape[0]
    hid = att_src1.shape[1]
    d2 = W2.shape[1]

    eye = jnp.eye(heads, dtype=F32)
    As1 = jnp.pad((att_src1[:, :, None] * eye[:, None, :]).reshape(d1, heads),
                  ((0, 0), (0, 16 - heads)))
    Ad1 = jnp.pad((att_dst1[:, :, None] * eye[:, None, :]).reshape(d1, heads),
                  ((0, 0), (0, 16 - heads)))
    As2 = jnp.pad(att_src2.T, ((0, 0), (0, 15)))
    Ad2 = jnp.pad(att_dst2.T, ((0, 0), (0, 15)))
    R1 = jnp.pad(jnp.repeat(jnp.eye(heads, dtype=F32), hid, axis=1),
                 ((0, 16 - heads), (0, 0)))
    R2 = jnp.zeros((16, d2), F32).at[0].set(1.0)

    npad = ((n + 8 * _NS - 1) // (8 * _NS)) * (8 * _NS)
    rpt = npad // _NS
    zd1 = jnp.zeros((rpt, d1), F32)
    zd2 = jnp.zeros((rpt, d2), F32)
    z16 = jnp.zeros((rpt, 16), F32)

    bn = 1000
    grid = (n // bn,)

    h1, as1, ad1 = pl.pallas_call(
        _tc_a,
        grid=grid,
        in_specs=[pl.BlockSpec((bn, d_in), lambda i: (i, 0)),
                  pl.BlockSpec((d_in, d1), lambda i: (0, 0)),
                  pl.BlockSpec((d1, 16), lambda i: (0, 0)),
                  pl.BlockSpec((d1, 16), lambda i: (0, 0))],
        out_specs=[pl.BlockSpec((bn, d1), lambda i: (i, 0)),
                   pl.BlockSpec((bn, 16), lambda i: (i, 0)),
                   pl.BlockSpec((bn, 16), lambda i: (i, 0))],
        out_shape=[jax.ShapeDtypeStruct((n, d1), F32),
                   jax.ShapeDtypeStruct((n, 16), F32),
                   jax.ShapeDtypeStruct((n, 16), F32)],
    )(x, W1, As1, Ad1)

    o1p, d1p = _make_edge_kernel(n, npad, e, d1, heads)(
        edge_index, h1, as1, ad1, zd1, z16)

    h2, as2, ad2 = pl.pallas_call(
        _tc_b,
        grid=grid,
        in_specs=[pl.BlockSpec((_NC, bn, d1), lambda i: (0, i, 0)),
                  pl.BlockSpec((_NC, bn, 16), lambda i: (0, i, 0)),
                  pl.BlockSpec((16, d1), lambda i: (0, 0)),
                  pl.BlockSpec((1, d1), lambda i: (0, 0)),
                  pl.BlockSpec((d1, d2), lambda i: (0, 0)),
                  pl.BlockSpec((d2, 16), lambda i: (0, 0)),
                  pl.BlockSpec((d2, 16), lambda i: (0, 0))],
        out_specs=[pl.BlockSpec((bn, d2), lambda i: (i, 0)),
                   pl.BlockSpec((bn, 16), lambda i: (i, 0)),
                   pl.BlockSpec((bn, 16), lambda i: (i, 0))],
        out_shape=[jax.ShapeDtypeStruct((n, d2), F32),
                   jax.ShapeDtypeStruct((n, 16), F32),
                   jax.ShapeDtypeStruct((n, 16), F32)],
    )(o1p, d1p, R1, b1.reshape(1, d1), W2, As2, Ad2)

    o2p, d2p = _make_edge_kernel(n, npad, e, d2, 1)(
        edge_index, h2, as2, ad2, zd2, z16)

    out = pl.pallas_call(
        _tc_c,
        grid=grid,
        in_specs=[pl.BlockSpec((_NC, bn, d2), lambda i: (0, i, 0)),
                  pl.BlockSpec((_NC, bn, 16), lambda i: (0, i, 0)),
                  pl.BlockSpec((16, d2), lambda i: (0, 0)),
                  pl.BlockSpec((1, d2), lambda i: (0, 0))],
        out_specs=pl.BlockSpec((bn, d2), lambda i: (i, 0)),
        out_shape=jax.ShapeDtypeStruct((n, d2), F32),
    )(o2p, d2p, R2, b2.reshape(1, d2))

    return out
